# baseline (speedup 1.0000x reference)
import jax
import jax.numpy as jnp
from jax.experimental import pallas as pl


def _leaky(x):
    return jnp.maximum(x, 0.01 * x)


def _final_body(x0_ref, h_ref, wla_ref, wlb_ref, bl_ref, o_ref):
    z = (jnp.sum(x0_ref[...] * wla_ref[...], axis=1, keepdims=True)
         + jnp.sum(h_ref[...] * wlb_ref[...], axis=1, keepdims=True) + bl_ref[...])
    o_ref[...] = jnp.maximum(z, 0.0) + jnp.log1p(jnp.exp(-jnp.abs(z)))


def kernel(node_features, edge_features, W11, b11, W12, b12, W21, b21, W22, b22, W31, b31, W32, b32, Wl, bl, edge_index):
    B, N, F = node_features.shape
    x0 = jnp.trunc(node_features[0])
    ei = edge_index[0]
    ea = edge_features[0]
    src, dst = ei[0], ei[1]

    def conv(x, W1, b1, W2, b2):
        tmp = jnp.concatenate([x[dst], x[src], ea], axis=1)
        h = _leaky(tmp @ W1.T + b1)
        m = h @ W2.T + b2
        agg = jax.ops.segment_min(m, dst, num_segments=N)
        cnt = jax.ops.segment_sum(jnp.ones((dst.shape[0], 1), m.dtype), dst, num_segments=N)
        return jnp.where(cnt > 0, agg, jnp.zeros_like(agg))

    h = _leaky(conv(x0, W11, b11, W12, b12))
    h = _leaky(conv(h, W21, b21, W22, b22))
    h = _leaky(conv(h, W31, b31, W32, b32))

    BN = 10000
    wla = Wl[:, :F]
    wlb = Wl[:, F:]
    out = pl.pallas_call(
        _final_body,
        grid=(N // BN,),
        in_specs=[
            pl.BlockSpec((BN, F), lambda i: (i, 0)),
            pl.BlockSpec((BN, 8), lambda i: (i, 0)),
            pl.BlockSpec(wla.shape, lambda i: (0, 0)),
            pl.BlockSpec(wlb.shape, lambda i: (0, 0)),
            pl.BlockSpec((1, 1), lambda i: (0, 0)),
        ],
        out_specs=pl.BlockSpec((BN, 1), lambda i: (i, 0)),
        out_shape=jax.ShapeDtypeStruct((N, 1), jnp.float32),
    )(x0, h, wla, wlb, bl.reshape(1, 1))
    return out.reshape(B, N, 1)


# speedup vs baseline: 6.2404x; 6.2404x over previous
import functools

import jax
import jax.numpy as jnp
from jax import lax
from jax.experimental import pallas as pl
from jax.experimental.pallas import tpu as pltpu
from jax.experimental.pallas import tpu_sc as plsc

N = 100000
E = 6400000
NT = 32
RNG = 3125
CHUNK = E // NT
PW = 4000
WE = 2048
EP = E + 8 * 1024 + 4160
GUARD = 2112
TROW = RNG
TFL = (RNG + 1) * 8

MESH = plsc.VectorSubcoreMesh(core_axis_name="c", subcore_axis_name="s")
CP = pltpu.CompilerParams(needs_layout_passes=False, use_tc_tiling_on_sc=False)


def _scal(ref, i):
    v = ref[pl.ds((i // 16) * 16, 16)]
    return v[i % 16]


def _extract(vec, idx16):
    iota = lax.iota(jnp.int32, 16)
    return jnp.sum(jnp.where(iota == idx16, vec, 0))

def _wid():
    return lax.axis_index("s") * 2 + lax.axis_index("c")


def _bucket(d):
    q = (d.astype(jnp.float32) * (1.0 / 3125.0)).astype(jnp.int32)
    rr = d - q * 3125
    q = q + jnp.where(rr >= 3125, 1, 0) - jnp.where(rr < 0, 1, 0)
    return q


def _zero32(ref):
    ref[pl.ds(0, 16)] = jnp.zeros((16,), jnp.int32)
    ref[pl.ds(16, 16)] = jnp.zeros((16,), jnp.int32)


def _offsets(cn_v, t):
    zero = jnp.zeros((16,), jnp.int32)

    def acc(tp, carry):
        s0, s1, p0, p1 = carry
        row0 = cn_v[pl.ds(tp * 32, 16)]
        row1 = cn_v[pl.ds(tp * 32 + 16, 16)]
        q0 = jnp.bitwise_and(row0 + 7, -8)
        q1 = jnp.bitwise_and(row1 + 7, -8)
        before = tp < t
        p0 = p0 + jnp.where(before, q0, 0)
        p1 = p1 + jnp.where(before, q1, 0)
        return (s0 + q0, s1 + q1, p0, p1)

    seg0, seg1, par0, par1 = lax.fori_loop(0, NT, acc, (zero, zero, zero, zero))
    ex0 = plsc.cumsum(seg0) - seg0
    ex1 = plsc.cumsum(seg1) - seg1 + jnp.sum(seg0)
    return ex0, ex1, seg0, seg1, par0, par1


@functools.partial(
    pl.kernel, mesh=MESH, compiler_params=CP,
    out_type=jax.ShapeDtypeStruct((NT * NT,), jnp.int32),
    scratch_types=[pltpu.VMEM((PW,), jnp.int32), pltpu.VMEM((32,), jnp.int32),
                   pltpu.SemaphoreType.DMA],
)
def _prep_a(dst_hbm, cnts, win_v, cnt_v, sem):
    t = _wid()
    _zero32(cnt_v)
    base = t * CHUNK

    def win(w, _):
        st = pl.multiple_of(base + w * PW, 8)
        pltpu.sync_copy(dst_hbm.at[pl.ds(st, PW)], win_v)

        def step(i, _):
            d = win_v[pl.ds(i * 16, 16)]
            q = _bucket(d)
            plsc.addupdate_scatter(cnt_v, [q], jnp.ones((16,), jnp.int32))
            return 0

        return lax.fori_loop(0, PW // 16, step, 0)

    lax.fori_loop(0, CHUNK // PW, win, 0)
    pltpu.sync_copy(cnt_v, cnts.at[pl.ds(t * 32, 32)])


@functools.partial(
    pl.kernel, mesh=MESH, compiler_params=CP,
    out_type=[jax.ShapeDtypeStruct((EP,), jnp.int32),
              jax.ShapeDtypeStruct((EP,), jnp.int32),
              jax.ShapeDtypeStruct((EP,), jnp.float32)],
    scratch_types=[pltpu.VMEM((PW,), jnp.int32), pltpu.VMEM((PW,), jnp.int32),
                   pltpu.VMEM((PW,), jnp.float32), pltpu.VMEM((PW,), jnp.int32),
                   pltpu.VMEM((PW,), jnp.int32), pltpu.VMEM((32,), jnp.int32),
                   pltpu.VMEM((NT * NT,), jnp.int32), pltpu.VMEM((256,), jnp.int32),
                   pltpu.VMEM((256,), jnp.int32), pltpu.VMEM((256,), jnp.float32),
                   pltpu.VMEM((256,), jnp.int32),
                   pltpu.SemaphoreType.DMA],
)
def _prep_b(dst_hbm, src_hbm, ea_hbm, cnts_hbm, bdst, bsrc, bea,
            dwin, swin, eawin, pos_v, rel_v, fill_v, cn_v,
            pad_pos, pad_i, pad_f, pad_s, sem):
    t = _wid()
    pltpu.sync_copy(cnts_hbm, cn_v)
    ex0, ex1, seg0, seg1, par0, par1 = _offsets(cn_v, t)
    fill_v[pl.ds(0, 16)] = ex0 + par0
    fill_v[pl.ds(16, 16)] = ex1 + par1
    base = t * CHUNK

    def win(w, _):
        st = pl.multiple_of(base + w * PW, 8)
        pltpu.sync_copy(dst_hbm.at[pl.ds(st, PW)], dwin)
        pltpu.sync_copy(src_hbm.at[pl.ds(st, PW)], swin)
        pltpu.sync_copy(ea_hbm.at[pl.ds(st, PW)], eawin)

        def step(i, _):
            sl = pl.ds(i * 16, 16)
            d = dwin[sl]
            q = _bucket(d)
            rel = d - q * 3125
            rank, _lm = plsc.scan_count(q)
            b16 = plsc.load_gather(fill_v, [q])
            pos_v[sl] = b16 + rank - 1
            rel_v[sl] = rel
            plsc.addupdate_scatter(fill_v, [q], jnp.ones((16,), jnp.int32))
            return 0

        lax.fori_loop(0, PW // 16, step, 0)
        pltpu.async_copy(rel_v, bdst.at[pos_v], sem).wait()
        pltpu.async_copy(swin, bsrc.at[pos_v], sem).wait()
        pltpu.async_copy(eawin, bea.at[pos_v], sem).wait()
        return 0

    lax.fori_loop(0, CHUNK // PW, win, 0)

    iota = lax.iota(jnp.int32, 16)
    lane8 = jnp.bitwise_and(iota, 7)

    def padstep(j, _):
        ridx = j * 2 + jnp.where(iota >= 8, 1, 0)
        f16 = plsc.load_gather(fill_v, [ridx])
        c16 = plsc.load_gather(cn_v, [t * 32 + ridx])
        padc = jnp.bitwise_and(c16 + 7, -8) - c16
        pos = jnp.where(lane8 < padc, f16 + lane8, EP - 32 + iota)
        pad_pos[pl.ds(j * 16, 16)] = pos
        pad_i[pl.ds(j * 16, 16)] = jnp.full((16,), TROW, jnp.int32)
        pad_s[pl.ds(j * 16, 16)] = jnp.zeros((16,), jnp.int32)
        pad_f[pl.ds(j * 16, 16)] = jnp.zeros((16,), jnp.float32)
        return 0

    lax.fori_loop(0, 16, padstep, 0)
    pltpu.async_copy(pad_i, bdst.at[pad_pos], sem).wait()
    pltpu.async_copy(pad_s, bsrc.at[pad_pos], sem).wait()
    pltpu.async_copy(pad_f, bea.at[pad_pos], sem).wait()

    @pl.when(t == 0)
    def _():
        tu = pl.multiple_of(jnp.sum(seg0) + jnp.sum(seg1), 8)

        def gstep(i, _):
            sl = pl.ds(i * 16, 16)
            rel_v[sl] = jnp.full((16,), TROW, jnp.int32)
            swin[sl] = jnp.zeros((16,), jnp.int32)
            eawin[sl] = jnp.zeros((16,), jnp.float32)
            return 0

        lax.fori_loop(0, GUARD // 16, gstep, 0)
        pltpu.sync_copy(rel_v.at[pl.ds(0, GUARD)], bdst.at[pl.ds(tu, GUARD)])
        pltpu.sync_copy(swin.at[pl.ds(0, GUARD)], bsrc.at[pl.ds(tu, GUARD)])
        pltpu.sync_copy(eawin.at[pl.ds(0, GUARD)], bea.at[pl.ds(tu, GUARD)])


@functools.partial(
    pl.kernel, mesh=MESH, compiler_params=CP,
    out_type=[jax.ShapeDtypeStruct((N * 8,), jnp.float32),
              jax.ShapeDtypeStruct((N, 8), jnp.float32)],
    scratch_types=[pltpu.VMEM((TFL,), jnp.float32), pltpu.VMEM((TFL,), jnp.float32),
                   pltpu.VMEM((WE, 8), jnp.float32), pltpu.VMEM((144,), jnp.float32),
                   pltpu.SemaphoreType.DMA],
)
def _p1(nf_hbm, pw_hbm, a_hbm, b_hbm, xtab, atab, brow_buf, wtmp, sem):
    r = _wid()
    pltpu.sync_copy(pw_hbm, wtmp.at[pl.ds(0, 136)])
    w1a = [_scal(wtmp, i) for i in range(64)]
    w1b = [_scal(wtmp, 64 + i) for i in range(64)]
    b1 = [_scal(wtmp, 128 + i) for i in range(8)]
    st = pl.multiple_of(r * RNG * 8, 8)
    pltpu.sync_copy(nf_hbm.at[pl.ds(st, RNG * 8)], xtab.at[pl.ds(0, RNG * 8)])

    def tstep(i, _):
        sl = pl.ds(i * 16, 16)
        v = xtab[sl]
        xtab[sl] = v.astype(jnp.int32).astype(jnp.float32)
        return 0

    lax.fori_loop(0, TFL // 16, tstep, 0)
    _node_tables_2d(xtab, atab, brow_buf, b_hbm, a_hbm, r, w1a, w1b, b1)


@functools.partial(
    pl.kernel, mesh=MESH, compiler_params=CP,
    out_type=[jax.ShapeDtypeStruct((N * 8,), jnp.float32),
              jax.ShapeDtypeStruct((N * 8,), jnp.float32),
              jax.ShapeDtypeStruct((N, 8), jnp.float32)],
    scratch_types=[pltpu.VMEM((TFL,), jnp.float32),
                   pltpu.VMEM((TFL,), jnp.float32),
                   pltpu.VMEM((WE,), jnp.int32), pltpu.VMEM((WE,), jnp.int32),
                   pltpu.VMEM((WE,), jnp.float32), pltpu.VMEM((WE, 8), jnp.float32),
                   pltpu.VMEM((NT * NT,), jnp.int32),
                   pltpu.VMEM((224,), jnp.float32),
                   pltpu.SemaphoreType.DMA, pltpu.SemaphoreType.DMA],
)
def _edge(bdst, bsrc, bea, cnts_hbm, a_hbm, b_hbm, ew_hbm, pw_hbm,
          act_hbm, a2_hbm, b2_hbm,
          atab, htab, dwin, swin, eawin, brows, cn_v, wtmp,
          sem, semg):
    r = _wid()
    pltpu.sync_copy(ew_hbm, wtmp.at[pl.ds(0, 80)])
    pltpu.sync_copy(pw_hbm, wtmp.at[pl.ds(80, 136)])
    w2 = [_scal(wtmp, i) for i in range(64)]
    wc = [_scal(wtmp, 64 + i) for i in range(8)]
    b2 = [_scal(wtmp, 72 + i) for i in range(8)]
    w1a = [_scal(wtmp, 80 + i) for i in range(64)]
    w1b = [_scal(wtmp, 144 + i) for i in range(64)]
    b1 = [_scal(wtmp, 208 + i) for i in range(8)]

    ast = pl.multiple_of(r * RNG * 8, 8)
    pltpu.sync_copy(a_hbm.at[pl.ds(ast, RNG * 8)], atab.at[pl.ds(0, RNG * 8)])

    def istep(i, _):
        htab[pl.ds(i * 16, 16)] = jnp.full((16,), jnp.inf, jnp.float32)
        return 0

    lax.fori_loop(0, TFL // 16, istep, 0)

    pltpu.sync_copy(cnts_hbm, cn_v)
    ex0, ex1, seg0, seg1, _p0, _p1v = _offsets(cn_v, r)
    r15 = jnp.bitwise_and(r, 15)
    ss = pl.multiple_of(_extract(jnp.where(r < 16, ex0, ex1), r15), 8)
    slen = _extract(jnp.where(r < 16, seg0, seg1), r15)
    nwin = (slen + WE - 1) // WE

    iota = lax.iota(jnp.int32, 16)
    iota8 = iota * 8

    def win(w, _):
        wst = pl.multiple_of(ss + w * WE, 8)
        pltpu.sync_copy(bdst.at[pl.ds(wst, WE)], dwin)
        pltpu.sync_copy(bsrc.at[pl.ds(wst, WE)], swin)
        pltpu.sync_copy(bea.at[pl.ds(wst, WE)], eawin)
        pltpu.async_copy(b_hbm.at[swin], brows, semg).wait()

        def step(i, _):
            sl = pl.ds(i * 16, 16)
            rel = dwin[sl]
            eav = eawin[sl]
            msk = (iota + (w * WE + i * 16)) < slen
            i8 = rel * 8
            erow = iota + i * 16
            colz = jnp.zeros((16,), jnp.int32)
            a = [plsc.load_gather(atab, [i8 + f]) for f in range(8)]
            b = [plsc.load_gather(brows, [erow, colz + f]) for f in range(8)]
            m = []
            h = []
            for f in range(8):
                pre = a[f] + b[f] + eav * wc[f]
                h.append(jnp.maximum(pre, 0.01 * pre))
            for k in range(8):
                acc = jnp.full((16,), 0.0, jnp.float32) + b2[k]
                for f in range(8):
                    acc = acc + h[f] * w2[k * 8 + f]
                m.append(acc)
            rank, _lm = plsc.scan_count(rel)
            mrank = jnp.max(jnp.where(msk, rank, 0))

            def rmw(c, _):
                mc = jnp.logical_and(msk, rank == (c + 1))
                for k in range(8):
                    old = plsc.load_gather(htab, [i8 + k], mask=mc)
                    plsc.store_scatter(htab, [i8 + k], jnp.minimum(old, m[k]), mask=mc)
                return 0

            lax.fori_loop(0, mrank, rmw, 0)
            return 0

        lax.fori_loop(0, WE // 16, step, 0)
        return 0

    lax.fori_loop(0, nwin, win, 0)

    def astep(i, _):
        sl = pl.ds(i * 16, 16)
        v = htab[sl]
        v = jnp.where(v == jnp.inf, 0.0, v)
        htab[sl] = jnp.maximum(v, 0.01 * v)
        return 0

    lax.fori_loop(0, TFL // 16, astep, 0)
    pltpu.sync_copy(htab.at[pl.ds(0, RNG * 8)], act_hbm.at[pl.ds(ast, RNG * 8)])

    _node_tables_2d(htab, atab, brows, b2_hbm, a2_hbm, r, w1a, w1b, b1)


def _node_tables_2d(src_tab, atab, brow2d, b_hbm, a_hbm, r, w1a, w1b, b1):
    iota = lax.iota(jnp.int32, 16)
    half = 1568

    for p in range(2):
        nbase = p * half
        nrows = half if p == 0 else RNG - half
        nsteps = (nrows + 15) // 16

        def pstep(i, _):
            n = iota + nbase + i * 16
            nc = jnp.minimum(n, RNG)
            msk = n < (nbase + nrows)
            n8 = nc * 8
            actf = [plsc.load_gather(src_tab, [n8 + f]) for f in range(8)]
            nrel = nc - nbase
            for g in range(8):
                ag = jnp.full((16,), 0.0, jnp.float32) + b1[g]
                bg = jnp.zeros((16,), jnp.float32)
                for f in range(8):
                    ag = ag + actf[f] * w1a[g * 8 + f]
                    bg = bg + actf[f] * w1b[g * 8 + f]
                plsc.store_scatter(atab, [n8 + g], ag, mask=msk)
                plsc.store_scatter(brow2d, [nrel, jnp.full((16,), g, jnp.int32)], bg, mask=msk)
            return 0

        lax.fori_loop(0, nsteps, pstep, 0)
        dstst = pl.multiple_of(r * RNG + nbase, 1)
        pltpu.sync_copy(brow2d.at[pl.ds(0, nrows)], b_hbm.at[pl.ds(dstst, nrows)])

    ast = pl.multiple_of(r * RNG * 8, 8)
    pltpu.sync_copy(atab.at[pl.ds(0, RNG * 8)], a_hbm.at[pl.ds(ast, RNG * 8)])


def _final_body(x_ref, h_ref, wla_ref, wlb_ref, bl_ref, o_ref):
    x0 = jnp.trunc(x_ref[...])
    z = (jnp.sum(x0 * wla_ref[...], axis=1, keepdims=True)
         + jnp.sum(h_ref[...] * wlb_ref[...], axis=1, keepdims=True) + bl_ref[...])
    o_ref[...] = jnp.maximum(z, 0.0) + jnp.log1p(jnp.exp(-jnp.abs(z)))


def kernel(node_features, edge_features, W11, b11, W12, b12, W21, b21, W22, b22,
           W31, b31, W32, b32, Wl, bl, edge_index):
    B, n, F = node_features.shape
    nf = node_features[0]
    src = edge_index[0, 0]
    dst = edge_index[0, 1]
    eaf = edge_features[0, :, 0]

    def pw(W1, b1):
        return jnp.concatenate([W1[:, :8].reshape(-1), W1[:, 8:16].reshape(-1), b1])

    def ew(W1, W2, b2):
        return jnp.concatenate([W2.reshape(-1), W1[:, 16], b2])

    pw1 = pw(W11, b11)
    pw2 = pw(W21, b21)
    pw3 = pw(W31, b31)
    ew1 = ew(W11, W12, b12)
    ew2 = ew(W21, W22, b22)
    ew3 = ew(W31, W32, b32)

    cnts = _prep_a(dst)
    bdst, bsrc, bea = _prep_b(dst, src, eaf, cnts)
    a1, b1t = _p1(nf.reshape(-1), pw1)
    _act1, a2, b2t = _edge(bdst, bsrc, bea, cnts, a1, b1t, ew1, pw2)
    _act2, a3, b3t = _edge(bdst, bsrc, bea, cnts, a2, b2t, ew2, pw3)
    act3, _a4, _b4 = _edge(bdst, bsrc, bea, cnts, a3, b3t, ew3, pw3)

    BN = 10000
    wla = Wl[:, :F]
    wlb = Wl[:, F:]
    out = pl.pallas_call(
        _final_body,
        grid=(N // BN,),
        in_specs=[
            pl.BlockSpec((BN, F), lambda i: (i, 0)),
            pl.BlockSpec((BN, 8), lambda i: (i, 0)),
            pl.BlockSpec(wla.shape, lambda i: (0, 0)),
            pl.BlockSpec(wlb.shape, lambda i: (0, 0)),
            pl.BlockSpec((1, 1), lambda i: (0, 0)),
        ],
        out_specs=pl.BlockSpec((BN, 1), lambda i: (i, 0)),
        out_shape=jax.ShapeDtypeStruct((N, 1), jnp.float32),
    )(nf, act3.reshape(N, 8), wla, wlb, bl.reshape(1, 1))
    return out.reshape(B, n, 1)


# speedup vs baseline: 21.2178x; 3.4001x over previous
import functools

import jax
import jax.numpy as jnp
from jax import lax
from jax.experimental import pallas as pl
from jax.experimental.pallas import tpu as pltpu
from jax.experimental.pallas import tpu_sc as plsc

N = 100000
E = 6400000
NT = 32
RNG = 3125
CHUNK = E // NT
PW = 4000
WE = 2048
EP = E + 8 * 1024 + 4160
GUARD = 2112
TROW = RNG
TFL = (RNG + 1) * 8

MESH = plsc.VectorSubcoreMesh(core_axis_name="c", subcore_axis_name="s")
CP = pltpu.CompilerParams(needs_layout_passes=False, use_tc_tiling_on_sc=False)


def _scal(ref, i):
    v = ref[pl.ds((i // 16) * 16, 16)]
    return v[i % 16]


def _extract(vec, idx16):
    iota = lax.iota(jnp.int32, 16)
    return jnp.sum(jnp.where(iota == idx16, vec, 0))

def _wid():
    return lax.axis_index("s") * 2 + lax.axis_index("c")


def _bucket(d):
    q = (d.astype(jnp.float32) * (1.0 / 3125.0)).astype(jnp.int32)
    rr = d - q * 3125
    q = q + jnp.where(rr >= 3125, 1, 0) - jnp.where(rr < 0, 1, 0)
    return q


def _zero32(ref):
    ref[pl.ds(0, 16)] = jnp.zeros((16,), jnp.int32)
    ref[pl.ds(16, 16)] = jnp.zeros((16,), jnp.int32)


def _offsets(cn_v, t):
    zero = jnp.zeros((16,), jnp.int32)

    def acc(tp, carry):
        s0, s1, p0, p1 = carry
        row0 = cn_v[pl.ds(tp * 32, 16)]
        row1 = cn_v[pl.ds(tp * 32 + 16, 16)]
        q0 = jnp.bitwise_and(row0 + 7, -8)
        q1 = jnp.bitwise_and(row1 + 7, -8)
        before = tp < t
        p0 = p0 + jnp.where(before, q0, 0)
        p1 = p1 + jnp.where(before, q1, 0)
        return (s0 + q0, s1 + q1, p0, p1)

    seg0, seg1, par0, par1 = lax.fori_loop(0, NT, acc, (zero, zero, zero, zero))
    ex0 = plsc.cumsum(seg0) - seg0
    ex1 = plsc.cumsum(seg1) - seg1 + jnp.sum(seg0)
    return ex0, ex1, seg0, seg1, par0, par1


@functools.partial(
    pl.kernel, mesh=MESH, compiler_params=CP,
    out_type=jax.ShapeDtypeStruct((NT * NT,), jnp.int32),
    scratch_types=[pltpu.VMEM((PW,), jnp.int32), pltpu.VMEM((32,), jnp.int32),
                   pltpu.SemaphoreType.DMA],
)
def _prep_a(dst_hbm, cnts, win_v, cnt_v, sem):
    t = _wid()
    _zero32(cnt_v)
    base = t * CHUNK

    def win(w, _):
        st = pl.multiple_of(base + w * PW, 8)
        pltpu.sync_copy(dst_hbm.at[pl.ds(st, PW)], win_v)

        def step(i, _):
            d = win_v[pl.ds(i * 16, 16)]
            q = _bucket(d)
            plsc.addupdate_scatter(cnt_v, [q], jnp.ones((16,), jnp.int32))
            return 0

        return lax.fori_loop(0, PW // 16, step, 0)

    lax.fori_loop(0, CHUNK // PW, win, 0)
    pltpu.sync_copy(cnt_v, cnts.at[pl.ds(t * 32, 32)])


FB = 1024
BUFW = FB + 16

@functools.partial(
    pl.kernel, mesh=MESH, compiler_params=CP,
    out_type=[jax.ShapeDtypeStruct((EP,), jnp.int32),
              jax.ShapeDtypeStruct((EP,), jnp.int32),
              jax.ShapeDtypeStruct((EP,), jnp.float32)],
    scratch_types=[pltpu.VMEM((PW,), jnp.int32), pltpu.VMEM((PW,), jnp.int32),
                   pltpu.VMEM((PW,), jnp.float32),
                   pltpu.VMEM((32 * BUFW,), jnp.int32),
                   pltpu.VMEM((32 * BUFW,), jnp.int32),
                   pltpu.VMEM((32 * BUFW,), jnp.float32),
                   pltpu.VMEM((32,), jnp.int32), pltpu.VMEM((32,), jnp.int32),
                   pltpu.VMEM((NT * NT,), jnp.int32),
                   pltpu.SemaphoreType.DMA],
)
def _prep_b(dst_hbm, src_hbm, ea_hbm, cnts_hbm, bdst, bsrc, bea,
            dwin, swin, eawin, bufd, bufs, bufe, fill_v, hp_v, cn_v, sem):
    t = _wid()
    pltpu.sync_copy(cnts_hbm, cn_v)
    ex0, ex1, seg0, seg1, par0, par1 = _offsets(cn_v, t)
    hp_v[pl.ds(0, 16)] = ex0 + par0
    hp_v[pl.ds(16, 16)] = ex1 + par1
    _zero32(fill_v)
    base = t * CHUNK
    iota = lax.iota(jnp.int32, 16)

    def flush_bucket(b, thresh):
        f0 = fill_v[pl.ds(0, 16)]
        f1 = fill_v[pl.ds(16, 16)]
        fb = _extract(jnp.where(b < 16, f0, f1), jnp.bitwise_and(b, 15))

        @pl.when(fb >= thresh)
        def _():
            h0 = hp_v[pl.ds(0, 16)]
            h1 = hp_v[pl.ds(16, 16)]
            hb = pl.multiple_of(
                _extract(jnp.where(b < 16, h0, h1), jnp.bitwise_and(b, 15)), 8)
            bb = pl.multiple_of(b * BUFW, 16)
            pltpu.sync_copy(bufd.at[pl.ds(bb, FB)], bdst.at[pl.ds(hb, FB)])
            pltpu.sync_copy(bufs.at[pl.ds(bb, FB)], bsrc.at[pl.ds(hb, FB)])
            pltpu.sync_copy(bufe.at[pl.ds(bb, FB)], bea.at[pl.ds(hb, FB)])
            bufd[pl.ds(bb, 16)] = bufd[pl.ds(bb + FB, 16)]
            bufs[pl.ds(bb, 16)] = bufs[pl.ds(bb + FB, 16)]
            bufe[pl.ds(bb, 16)] = bufe[pl.ds(bb + FB, 16)]
            fill_v[pl.ds(0, 16)] = jnp.where(iota == b, f0 - FB, f0)
            fill_v[pl.ds(16, 16)] = jnp.where(iota + 16 == b, f1 - FB, f1)
            hp_v[pl.ds(0, 16)] = jnp.where(iota == b, h0 + FB, h0)
            hp_v[pl.ds(16, 16)] = jnp.where(iota + 16 == b, h1 + FB, h1)

    def win(w, _):
        st = pl.multiple_of(base + w * PW, 8)
        pltpu.sync_copy(dst_hbm.at[pl.ds(st, PW)], dwin)
        pltpu.sync_copy(src_hbm.at[pl.ds(st, PW)], swin)
        pltpu.sync_copy(ea_hbm.at[pl.ds(st, PW)], eawin)

        def step(i, _):
            sl = pl.ds(i * 16, 16)
            d = dwin[sl]
            q = _bucket(d)
            rel = d - q * 3125
            rank, _lm = plsc.scan_count(q)
            f16 = plsc.load_gather(fill_v, [q])
            pos = q * BUFW + f16 + rank - 1
            plsc.store_scatter(bufd, [pos], rel)
            plsc.store_scatter(bufs, [pos], swin[sl])
            plsc.store_scatter(bufe, [pos], eawin[sl])
            plsc.addupdate_scatter(fill_v, [q], jnp.ones((16,), jnp.int32))
            f0 = fill_v[pl.ds(0, 16)]
            f1 = fill_v[pl.ds(16, 16)]
            mx = jnp.max(jnp.maximum(f0, f1))

            @pl.when(mx >= FB)
            def _():
                lax.fori_loop(0, 32, lambda b, c: (flush_bucket(b, FB), c)[1], 0)

            return 0

        lax.fori_loop(0, PW // 16, step, 0)
        return 0

    lax.fori_loop(0, CHUNK // PW, win, 0)

    def drain_bucket(b, _):
        f0 = fill_v[pl.ds(0, 16)]
        f1 = fill_v[pl.ds(16, 16)]
        fb = _extract(jnp.where(b < 16, f0, f1), jnp.bitwise_and(b, 15))
        plen = jnp.bitwise_and(fb + 7, -8)
        bb = pl.multiple_of(b * BUFW, 16)
        wst = jnp.bitwise_and(fb, -16)
        lanes = wst + iota
        pm = jnp.logical_and(lanes >= fb, lanes < plen)
        vd = bufd[pl.ds(bb + wst, 16)]
        bufd[pl.ds(bb + wst, 16)] = jnp.where(pm, TROW, vd)
        vs = bufs[pl.ds(bb + wst, 16)]
        bufs[pl.ds(bb + wst, 16)] = jnp.where(pm, 0, vs)
        ve = bufe[pl.ds(bb + wst, 16)]
        bufe[pl.ds(bb + wst, 16)] = jnp.where(pm, 0.0, ve)
        h0 = hp_v[pl.ds(0, 16)]
        h1 = hp_v[pl.ds(16, 16)]
        hb = pl.multiple_of(
            _extract(jnp.where(b < 16, h0, h1), jnp.bitwise_and(b, 15)), 8)

        def d128(j, _):
            @pl.when(plen - j * 128 >= 128)
            def _():
                o = pl.multiple_of(j * 128, 8)
                pltpu.sync_copy(bufd.at[pl.ds(bb + o, 128)], bdst.at[pl.ds(hb + o, 128)])
                pltpu.sync_copy(bufs.at[pl.ds(bb + o, 128)], bsrc.at[pl.ds(hb + o, 128)])
                pltpu.sync_copy(bufe.at[pl.ds(bb + o, 128)], bea.at[pl.ds(hb + o, 128)])
            return 0

        lax.fori_loop(0, (BUFW + 127) // 128, d128, 0)
        nfull = jnp.bitwise_and(plen, -128)

        def d8(j, _):
            @pl.when(nfull + j * 8 < plen)
            def _():
                o = pl.multiple_of(nfull + j * 8, 8)
                pltpu.sync_copy(bufd.at[pl.ds(bb + o, 8)], bdst.at[pl.ds(hb + o, 8)])
                pltpu.sync_copy(bufs.at[pl.ds(bb + o, 8)], bsrc.at[pl.ds(hb + o, 8)])
                pltpu.sync_copy(bufe.at[pl.ds(bb + o, 8)], bea.at[pl.ds(hb + o, 8)])
            return 0

        lax.fori_loop(0, 16, d8, 0)
        return 0

    lax.fori_loop(0, 32, drain_bucket, 0)

    @pl.when(t == 0)
    def _():
        tu = pl.multiple_of(jnp.sum(seg0) + jnp.sum(seg1), 8)

        def gstep(i, _):
            sl = pl.ds(i * 16, 16)
            dwin[sl] = jnp.full((16,), TROW, jnp.int32)
            swin[sl] = jnp.zeros((16,), jnp.int32)
            eawin[sl] = jnp.zeros((16,), jnp.float32)
            return 0

        lax.fori_loop(0, GUARD // 16, gstep, 0)
        pltpu.sync_copy(dwin.at[pl.ds(0, GUARD)], bdst.at[pl.ds(tu, GUARD)])
        pltpu.sync_copy(swin.at[pl.ds(0, GUARD)], bsrc.at[pl.ds(tu, GUARD)])
        pltpu.sync_copy(eawin.at[pl.ds(0, GUARD)], bea.at[pl.ds(tu, GUARD)])


@functools.partial(
    pl.kernel, mesh=MESH, compiler_params=CP,
    out_type=[jax.ShapeDtypeStruct((N * 8,), jnp.float32),
              jax.ShapeDtypeStruct((N, 8), jnp.float32)],
    scratch_types=[pltpu.VMEM((TFL,), jnp.float32), pltpu.VMEM((TFL,), jnp.float32),
                   pltpu.VMEM((WE, 8), jnp.float32), pltpu.VMEM((144,), jnp.float32),
                   pltpu.SemaphoreType.DMA],
)
def _p1(nf_hbm, pw_hbm, a_hbm, b_hbm, xtab, atab, brow_buf, wtmp, sem):
    r = _wid()
    pltpu.sync_copy(pw_hbm, wtmp.at[pl.ds(0, 136)])
    w1a = [_scal(wtmp, i) for i in range(64)]
    w1b = [_scal(wtmp, 64 + i) for i in range(64)]
    b1 = [_scal(wtmp, 128 + i) for i in range(8)]
    st = pl.multiple_of(r * RNG * 8, 8)
    pltpu.sync_copy(nf_hbm.at[pl.ds(st, RNG * 8)], xtab.at[pl.ds(0, RNG * 8)])

    def tstep(i, _):
        sl = pl.ds(i * 16, 16)
        v = xtab[sl]
        xtab[sl] = v.astype(jnp.int32).astype(jnp.float32)
        return 0

    lax.fori_loop(0, TFL // 16, tstep, 0)
    _node_tables_2d(xtab, atab, brow_buf, b_hbm, a_hbm, r, w1a, w1b, b1)


@functools.partial(
    pl.kernel, mesh=MESH, compiler_params=CP,
    out_type=[jax.ShapeDtypeStruct((N * 8,), jnp.float32),
              jax.ShapeDtypeStruct((N * 8,), jnp.float32),
              jax.ShapeDtypeStruct((N, 8), jnp.float32)],
    scratch_types=[pltpu.VMEM((TFL,), jnp.float32),
                   pltpu.VMEM((TFL,), jnp.float32),
                   pltpu.VMEM((WE,), jnp.int32), pltpu.VMEM((WE,), jnp.int32),
                   pltpu.VMEM((WE,), jnp.float32), pltpu.VMEM((WE, 8), jnp.float32),
                   pltpu.VMEM((NT * NT,), jnp.int32),
                   pltpu.VMEM((224,), jnp.float32),
                   pltpu.SemaphoreType.DMA, pltpu.SemaphoreType.DMA],
)
def _edge(bdst, bsrc, bea, cnts_hbm, a_hbm, b_hbm, ew_hbm, pw_hbm,
          act_hbm, a2_hbm, b2_hbm,
          atab, htab, dwin, swin, eawin, brows, cn_v, wtmp,
          sem, semg):
    r = _wid()
    pltpu.sync_copy(ew_hbm, wtmp.at[pl.ds(0, 80)])
    pltpu.sync_copy(pw_hbm, wtmp.at[pl.ds(80, 136)])
    w2 = [_scal(wtmp, i) for i in range(64)]
    wc = [_scal(wtmp, 64 + i) for i in range(8)]
    b2 = [_scal(wtmp, 72 + i) for i in range(8)]
    w1a = [_scal(wtmp, 80 + i) for i in range(64)]
    w1b = [_scal(wtmp, 144 + i) for i in range(64)]
    b1 = [_scal(wtmp, 208 + i) for i in range(8)]

    ast = pl.multiple_of(r * RNG * 8, 8)
    pltpu.sync_copy(a_hbm.at[pl.ds(ast, RNG * 8)], atab.at[pl.ds(0, RNG * 8)])

    def istep(i, _):
        htab[pl.ds(i * 16, 16)] = jnp.full((16,), jnp.inf, jnp.float32)
        return 0

    lax.fori_loop(0, TFL // 16, istep, 0)

    pltpu.sync_copy(cnts_hbm, cn_v)
    ex0, ex1, seg0, seg1, _p0, _p1v = _offsets(cn_v, r)
    r15 = jnp.bitwise_and(r, 15)
    ss = pl.multiple_of(_extract(jnp.where(r < 16, ex0, ex1), r15), 8)
    slen = _extract(jnp.where(r < 16, seg0, seg1), r15)
    nwin = (slen + WE - 1) // WE

    iota = lax.iota(jnp.int32, 16)
    iota8 = iota * 8

    def win(w, _):
        wst = pl.multiple_of(ss + w * WE, 8)
        pltpu.sync_copy(bdst.at[pl.ds(wst, WE)], dwin)
        pltpu.sync_copy(bsrc.at[pl.ds(wst, WE)], swin)
        pltpu.sync_copy(bea.at[pl.ds(wst, WE)], eawin)
        pltpu.async_copy(b_hbm.at[swin], brows, semg).wait()

        def step(i, _):
            sl = pl.ds(i * 16, 16)
            rel = dwin[sl]
            eav = eawin[sl]
            msk = (iota + (w * WE + i * 16)) < slen
            i8 = rel * 8
            erow = iota + i * 16
            colz = jnp.zeros((16,), jnp.int32)
            a = [plsc.load_gather(atab, [i8 + f]) for f in range(8)]
            b = [plsc.load_gather(brows, [erow, colz + f]) for f in range(8)]
            m = []
            h = []
            for f in range(8):
                pre = a[f] + b[f] + eav * wc[f]
                h.append(jnp.maximum(pre, 0.01 * pre))
            for k in range(8):
                acc = jnp.full((16,), 0.0, jnp.float32) + b2[k]
                for f in range(8):
                    acc = acc + h[f] * w2[k * 8 + f]
                m.append(acc)
            rank, _lm = plsc.scan_count(rel)
            mrank = jnp.max(jnp.where(msk, rank, 0))

            def rmw(c, _):
                mc = jnp.logical_and(msk, rank == (c + 1))
                for k in range(8):
                    old = plsc.load_gather(htab, [i8 + k], mask=mc)
                    plsc.store_scatter(htab, [i8 + k], jnp.minimum(old, m[k]), mask=mc)
                return 0

            lax.fori_loop(0, mrank, rmw, 0)
            return 0

        lax.fori_loop(0, WE // 16, step, 0)
        return 0

    lax.fori_loop(0, nwin, win, 0)

    def astep(i, _):
        sl = pl.ds(i * 16, 16)
        v = htab[sl]
        v = jnp.where(v == jnp.inf, 0.0, v)
        htab[sl] = jnp.maximum(v, 0.01 * v)
        return 0

    lax.fori_loop(0, TFL // 16, astep, 0)
    pltpu.sync_copy(htab.at[pl.ds(0, RNG * 8)], act_hbm.at[pl.ds(ast, RNG * 8)])

    _node_tables_2d(htab, atab, brows, b2_hbm, a2_hbm, r, w1a, w1b, b1)


def _node_tables_2d(src_tab, atab, brow2d, b_hbm, a_hbm, r, w1a, w1b, b1):
    iota = lax.iota(jnp.int32, 16)
    half = 1568

    for p in range(2):
        nbase = p * half
        nrows = half if p == 0 else RNG - half
        nsteps = (nrows + 15) // 16

        def pstep(i, _):
            n = iota + nbase + i * 16
            nc = jnp.minimum(n, RNG)
            msk = n < (nbase + nrows)
            n8 = nc * 8
            actf = [plsc.load_gather(src_tab, [n8 + f]) for f in range(8)]
            nrel = nc - nbase
            for g in range(8):
                ag = jnp.full((16,), 0.0, jnp.float32) + b1[g]
                bg = jnp.zeros((16,), jnp.float32)
                for f in range(8):
                    ag = ag + actf[f] * w1a[g * 8 + f]
                    bg = bg + actf[f] * w1b[g * 8 + f]
                plsc.store_scatter(atab, [n8 + g], ag, mask=msk)
                plsc.store_scatter(brow2d, [nrel, jnp.full((16,), g, jnp.int32)], bg, mask=msk)
            return 0

        lax.fori_loop(0, nsteps, pstep, 0)
        dstst = pl.multiple_of(r * RNG + nbase, 1)
        pltpu.sync_copy(brow2d.at[pl.ds(0, nrows)], b_hbm.at[pl.ds(dstst, nrows)])

    ast = pl.multiple_of(r * RNG * 8, 8)
    pltpu.sync_copy(atab.at[pl.ds(0, RNG * 8)], a_hbm.at[pl.ds(ast, RNG * 8)])


def _final_body(x_ref, h_ref, wla_ref, wlb_ref, bl_ref, o_ref):
    x0 = jnp.trunc(x_ref[...])
    z = (jnp.sum(x0 * wla_ref[...], axis=1, keepdims=True)
         + jnp.sum(h_ref[...] * wlb_ref[...], axis=1, keepdims=True) + bl_ref[...])
    o_ref[...] = jnp.maximum(z, 0.0) + jnp.log1p(jnp.exp(-jnp.abs(z)))


def kernel(node_features, edge_features, W11, b11, W12, b12, W21, b21, W22, b22,
           W31, b31, W32, b32, Wl, bl, edge_index):
    B, n, F = node_features.shape
    nf = node_features[0]
    src = edge_index[0, 0]
    dst = edge_index[0, 1]
    eaf = edge_features[0, :, 0]

    def pw(W1, b1):
        return jnp.concatenate([W1[:, :8].reshape(-1), W1[:, 8:16].reshape(-1), b1])

    def ew(W1, W2, b2):
        return jnp.concatenate([W2.reshape(-1), W1[:, 16], b2])

    pw1 = pw(W11, b11)
    pw2 = pw(W21, b21)
    pw3 = pw(W31, b31)
    ew1 = ew(W11, W12, b12)
    ew2 = ew(W21, W22, b22)
    ew3 = ew(W31, W32, b32)

    cnts = _prep_a(dst)
    bdst, bsrc, bea = _prep_b(dst, src, eaf, cnts)
    a1, b1t = _p1(nf.reshape(-1), pw1)
    _act1, a2, b2t = _edge(bdst, bsrc, bea, cnts, a1, b1t, ew1, pw2)
    _act2, a3, b3t = _edge(bdst, bsrc, bea, cnts, a2, b2t, ew2, pw3)
    act3, _a4, _b4 = _edge(bdst, bsrc, bea, cnts, a3, b3t, ew3, pw3)

    BN = 10000
    wla = Wl[:, :F]
    wlb = Wl[:, F:]
    out = pl.pallas_call(
        _final_body,
        grid=(N // BN,),
        in_specs=[
            pl.BlockSpec((BN, F), lambda i: (i, 0)),
            pl.BlockSpec((BN, 8), lambda i: (i, 0)),
            pl.BlockSpec(wla.shape, lambda i: (0, 0)),
            pl.BlockSpec(wlb.shape, lambda i: (0, 0)),
            pl.BlockSpec((1, 1), lambda i: (0, 0)),
        ],
        out_specs=pl.BlockSpec((BN, 1), lambda i: (i, 0)),
        out_shape=jax.ShapeDtypeStruct((N, 1), jnp.float32),
    )(nf, act3.reshape(N, 8), wla, wlb, bl.reshape(1, 1))
    return out.reshape(B, n, 1)


# speedup vs baseline: 24.1703x; 1.1392x over previous
import functools

import jax
import jax.numpy as jnp
from jax import lax
from jax.experimental import pallas as pl
from jax.experimental.pallas import tpu as pltpu
from jax.experimental.pallas import tpu_sc as plsc

N = 100000
E = 6400000
NT = 32
RNG = 3125
CHUNK = E // NT
PW = 4000
WE = 2048
EP = E + 8 * 1024 + 4160
GUARD = 2112
TROW = RNG
TFL = (RNG + 1) * 8

MESH = plsc.VectorSubcoreMesh(core_axis_name="c", subcore_axis_name="s")
CP = pltpu.CompilerParams(needs_layout_passes=False, use_tc_tiling_on_sc=False)


def _scal(ref, i):
    v = ref[pl.ds((i // 16) * 16, 16)]
    return v[i % 16]


def _extract(vec, idx16):
    iota = lax.iota(jnp.int32, 16)
    return jnp.sum(jnp.where(iota == idx16, vec, 0))

def _wid():
    return lax.axis_index("s") * 2 + lax.axis_index("c")


def _bucket(d):
    q = (d.astype(jnp.float32) * (1.0 / 3125.0)).astype(jnp.int32)
    rr = d - q * 3125
    q = q + jnp.where(rr >= 3125, 1, 0) - jnp.where(rr < 0, 1, 0)
    return q


def _zero32(ref):
    ref[pl.ds(0, 16)] = jnp.zeros((16,), jnp.int32)
    ref[pl.ds(16, 16)] = jnp.zeros((16,), jnp.int32)


def _offsets(cn_v, t):
    zero = jnp.zeros((16,), jnp.int32)

    def acc(tp, carry):
        s0, s1, p0, p1 = carry
        row0 = cn_v[pl.ds(tp * 32, 16)]
        row1 = cn_v[pl.ds(tp * 32 + 16, 16)]
        q0 = jnp.bitwise_and(row0 + 7, -8)
        q1 = jnp.bitwise_and(row1 + 7, -8)
        before = tp < t
        p0 = p0 + jnp.where(before, q0, 0)
        p1 = p1 + jnp.where(before, q1, 0)
        return (s0 + q0, s1 + q1, p0, p1)

    seg0, seg1, par0, par1 = lax.fori_loop(0, NT, acc, (zero, zero, zero, zero))
    ex0 = plsc.cumsum(seg0) - seg0
    ex1 = plsc.cumsum(seg1) - seg1 + jnp.sum(seg0)
    return ex0, ex1, seg0, seg1, par0, par1


@functools.partial(
    pl.kernel, mesh=MESH, compiler_params=CP,
    out_type=jax.ShapeDtypeStruct((NT * NT,), jnp.int32),
    scratch_types=[pltpu.VMEM((PW,), jnp.int32), pltpu.VMEM((32,), jnp.int32),
                   pltpu.SemaphoreType.DMA],
)
def _prep_a(dst_hbm, cnts, win_v, cnt_v, sem):
    t = _wid()
    _zero32(cnt_v)
    base = t * CHUNK

    def win(w, _):
        st = pl.multiple_of(base + w * PW, 8)
        pltpu.sync_copy(dst_hbm.at[pl.ds(st, PW)], win_v)

        def step(i, _):
            d = win_v[pl.ds(i * 16, 16)]
            q = _bucket(d)
            plsc.addupdate_scatter(cnt_v, [q], jnp.ones((16,), jnp.int32))
            return 0

        return lax.fori_loop(0, PW // 16, step, 0)

    lax.fori_loop(0, CHUNK // PW, win, 0)
    pltpu.sync_copy(cnt_v, cnts.at[pl.ds(t * 32, 32)])


FB = 1024
BUFW = FB + 16

@functools.partial(
    pl.kernel, mesh=MESH, compiler_params=CP,
    out_type=[jax.ShapeDtypeStruct((EP,), jnp.int32),
              jax.ShapeDtypeStruct((EP,), jnp.int32),
              jax.ShapeDtypeStruct((EP,), jnp.float32)],
    scratch_types=[pltpu.VMEM((PW,), jnp.int32), pltpu.VMEM((PW,), jnp.int32),
                   pltpu.VMEM((PW,), jnp.float32),
                   pltpu.VMEM((32 * BUFW,), jnp.int32),
                   pltpu.VMEM((32 * BUFW,), jnp.int32),
                   pltpu.VMEM((32 * BUFW,), jnp.float32),
                   pltpu.VMEM((32,), jnp.int32), pltpu.VMEM((32,), jnp.int32),
                   pltpu.VMEM((NT * NT,), jnp.int32),
                   pltpu.SemaphoreType.DMA],
)
def _prep_b(dst_hbm, src_hbm, ea_hbm, cnts_hbm, bdst, bsrc, bea,
            dwin, swin, eawin, bufd, bufs, bufe, fill_v, hp_v, cn_v, sem):
    t = _wid()
    pltpu.sync_copy(cnts_hbm, cn_v)
    ex0, ex1, seg0, seg1, par0, par1 = _offsets(cn_v, t)
    hp_v[pl.ds(0, 16)] = ex0 + par0
    hp_v[pl.ds(16, 16)] = ex1 + par1
    _zero32(fill_v)
    base = t * CHUNK
    iota = lax.iota(jnp.int32, 16)

    def flush_bucket(b, thresh):
        f0 = fill_v[pl.ds(0, 16)]
        f1 = fill_v[pl.ds(16, 16)]
        fb = _extract(jnp.where(b < 16, f0, f1), jnp.bitwise_and(b, 15))

        @pl.when(fb >= thresh)
        def _():
            h0 = hp_v[pl.ds(0, 16)]
            h1 = hp_v[pl.ds(16, 16)]
            hb = pl.multiple_of(
                _extract(jnp.where(b < 16, h0, h1), jnp.bitwise_and(b, 15)), 8)
            bb = pl.multiple_of(b * BUFW, 16)
            pltpu.sync_copy(bufd.at[pl.ds(bb, FB)], bdst.at[pl.ds(hb, FB)])
            pltpu.sync_copy(bufs.at[pl.ds(bb, FB)], bsrc.at[pl.ds(hb, FB)])
            pltpu.sync_copy(bufe.at[pl.ds(bb, FB)], bea.at[pl.ds(hb, FB)])
            bufd[pl.ds(bb, 16)] = bufd[pl.ds(bb + FB, 16)]
            bufs[pl.ds(bb, 16)] = bufs[pl.ds(bb + FB, 16)]
            bufe[pl.ds(bb, 16)] = bufe[pl.ds(bb + FB, 16)]
            fill_v[pl.ds(0, 16)] = jnp.where(iota == b, f0 - FB, f0)
            fill_v[pl.ds(16, 16)] = jnp.where(iota + 16 == b, f1 - FB, f1)
            hp_v[pl.ds(0, 16)] = jnp.where(iota == b, h0 + FB, h0)
            hp_v[pl.ds(16, 16)] = jnp.where(iota + 16 == b, h1 + FB, h1)

    def win(w, _):
        st = pl.multiple_of(base + w * PW, 8)
        pltpu.sync_copy(dst_hbm.at[pl.ds(st, PW)], dwin)
        pltpu.sync_copy(src_hbm.at[pl.ds(st, PW)], swin)
        pltpu.sync_copy(ea_hbm.at[pl.ds(st, PW)], eawin)

        def step(i, _):
            sl = pl.ds(i * 16, 16)
            d = dwin[sl]
            q = _bucket(d)
            rel = d - q * 3125
            rank, _lm = plsc.scan_count(q)
            f16 = plsc.load_gather(fill_v, [q])
            pos = q * BUFW + f16 + rank - 1
            plsc.store_scatter(bufd, [pos], rel)
            plsc.store_scatter(bufs, [pos], swin[sl])
            plsc.store_scatter(bufe, [pos], eawin[sl])
            plsc.addupdate_scatter(fill_v, [q], jnp.ones((16,), jnp.int32))
            f0 = fill_v[pl.ds(0, 16)]
            f1 = fill_v[pl.ds(16, 16)]
            mx = jnp.max(jnp.maximum(f0, f1))

            @pl.when(mx >= FB)
            def _():
                lax.fori_loop(0, 32, lambda b, c: (flush_bucket(b, FB), c)[1], 0)

            return 0

        lax.fori_loop(0, PW // 16, step, 0)
        return 0

    lax.fori_loop(0, CHUNK // PW, win, 0)

    def drain_bucket(b, _):
        f0 = fill_v[pl.ds(0, 16)]
        f1 = fill_v[pl.ds(16, 16)]
        fb = _extract(jnp.where(b < 16, f0, f1), jnp.bitwise_and(b, 15))
        plen = jnp.bitwise_and(fb + 7, -8)
        bb = pl.multiple_of(b * BUFW, 16)
        wst = jnp.bitwise_and(fb, -16)
        lanes = wst + iota
        pm = jnp.logical_and(lanes >= fb, lanes < plen)
        vd = bufd[pl.ds(bb + wst, 16)]
        bufd[pl.ds(bb + wst, 16)] = jnp.where(pm, TROW, vd)
        vs = bufs[pl.ds(bb + wst, 16)]
        bufs[pl.ds(bb + wst, 16)] = jnp.where(pm, 0, vs)
        ve = bufe[pl.ds(bb + wst, 16)]
        bufe[pl.ds(bb + wst, 16)] = jnp.where(pm, 0.0, ve)
        h0 = hp_v[pl.ds(0, 16)]
        h1 = hp_v[pl.ds(16, 16)]
        hb = pl.multiple_of(
            _extract(jnp.where(b < 16, h0, h1), jnp.bitwise_and(b, 15)), 8)

        def d128(j, _):
            @pl.when(plen - j * 128 >= 128)
            def _():
                o = pl.multiple_of(j * 128, 8)
                pltpu.sync_copy(bufd.at[pl.ds(bb + o, 128)], bdst.at[pl.ds(hb + o, 128)])
                pltpu.sync_copy(bufs.at[pl.ds(bb + o, 128)], bsrc.at[pl.ds(hb + o, 128)])
                pltpu.sync_copy(bufe.at[pl.ds(bb + o, 128)], bea.at[pl.ds(hb + o, 128)])
            return 0

        lax.fori_loop(0, (BUFW + 127) // 128, d128, 0)
        nfull = jnp.bitwise_and(plen, -128)

        def d8(j, _):
            @pl.when(nfull + j * 8 < plen)
            def _():
                o = pl.multiple_of(nfull + j * 8, 8)
                pltpu.sync_copy(bufd.at[pl.ds(bb + o, 8)], bdst.at[pl.ds(hb + o, 8)])
                pltpu.sync_copy(bufs.at[pl.ds(bb + o, 8)], bsrc.at[pl.ds(hb + o, 8)])
                pltpu.sync_copy(bufe.at[pl.ds(bb + o, 8)], bea.at[pl.ds(hb + o, 8)])
            return 0

        lax.fori_loop(0, 16, d8, 0)
        return 0

    lax.fori_loop(0, 32, drain_bucket, 0)

    @pl.when(t == 0)
    def _():
        tu = pl.multiple_of(jnp.sum(seg0) + jnp.sum(seg1), 8)

        def gstep(i, _):
            sl = pl.ds(i * 16, 16)
            dwin[sl] = jnp.full((16,), TROW, jnp.int32)
            swin[sl] = jnp.zeros((16,), jnp.int32)
            eawin[sl] = jnp.zeros((16,), jnp.float32)
            return 0

        lax.fori_loop(0, GUARD // 16, gstep, 0)
        pltpu.sync_copy(dwin.at[pl.ds(0, GUARD)], bdst.at[pl.ds(tu, GUARD)])
        pltpu.sync_copy(swin.at[pl.ds(0, GUARD)], bsrc.at[pl.ds(tu, GUARD)])
        pltpu.sync_copy(eawin.at[pl.ds(0, GUARD)], bea.at[pl.ds(tu, GUARD)])


@functools.partial(
    pl.kernel, mesh=MESH, compiler_params=CP,
    out_type=[jax.ShapeDtypeStruct((N * 8,), jnp.float32),
              jax.ShapeDtypeStruct((N, 8), jnp.float32)],
    scratch_types=[pltpu.VMEM((TFL,), jnp.float32), pltpu.VMEM((TFL,), jnp.float32),
                   pltpu.VMEM((WE, 8), jnp.float32), pltpu.VMEM((144,), jnp.float32),
                   pltpu.SemaphoreType.DMA],
)
def _p1(nf_hbm, pw_hbm, a_hbm, b_hbm, xtab, atab, brow_buf, wtmp, sem):
    r = _wid()
    pltpu.sync_copy(pw_hbm, wtmp.at[pl.ds(0, 136)])
    w1a = [_scal(wtmp, i) for i in range(64)]
    w1b = [_scal(wtmp, 64 + i) for i in range(64)]
    b1 = [_scal(wtmp, 128 + i) for i in range(8)]
    st = pl.multiple_of(r * RNG * 8, 8)
    pltpu.sync_copy(nf_hbm.at[pl.ds(st, RNG * 8)], xtab.at[pl.ds(0, RNG * 8)])

    def tstep(i, _):
        sl = pl.ds(i * 16, 16)
        v = xtab[sl]
        xtab[sl] = v.astype(jnp.int32).astype(jnp.float32)
        return 0

    lax.fori_loop(0, TFL // 16, tstep, 0)
    _node_tables_2d(xtab, atab, brow_buf, b_hbm, a_hbm, r, w1a, w1b, b1)


@functools.partial(
    pl.kernel, mesh=MESH, compiler_params=CP,
    out_type=[jax.ShapeDtypeStruct((N * 8,), jnp.float32),
              jax.ShapeDtypeStruct((N * 8,), jnp.float32),
              jax.ShapeDtypeStruct((N, 8), jnp.float32)],
    scratch_types=[pltpu.VMEM((TFL,), jnp.float32),
                   pltpu.VMEM((TFL,), jnp.float32),
                   pltpu.VMEM((WE,), jnp.int32), pltpu.VMEM((WE,), jnp.int32),
                   pltpu.VMEM((WE,), jnp.float32), pltpu.VMEM((WE, 8), jnp.float32),
                   pltpu.VMEM((NT * NT,), jnp.int32),
                   pltpu.VMEM((224,), jnp.float32),
                   pltpu.SemaphoreType.DMA, pltpu.SemaphoreType.DMA],
)
def _edge(bdst, bsrc, bea, cnts_hbm, a_hbm, b_hbm, ew_hbm, pw_hbm,
          act_hbm, a2_hbm, b2_hbm,
          atab, htab, dwin, swin, eawin, brows, cn_v, wtmp,
          sem, semg):
    r = _wid()
    pltpu.sync_copy(ew_hbm, wtmp.at[pl.ds(0, 80)])
    pltpu.sync_copy(pw_hbm, wtmp.at[pl.ds(80, 136)])
    w2 = [_scal(wtmp, i) for i in range(64)]
    wc = [_scal(wtmp, 64 + i) for i in range(8)]
    b2 = [_scal(wtmp, 72 + i) for i in range(8)]
    w1a = [_scal(wtmp, 80 + i) for i in range(64)]
    w1b = [_scal(wtmp, 144 + i) for i in range(64)]
    b1 = [_scal(wtmp, 208 + i) for i in range(8)]

    ast = pl.multiple_of(r * RNG * 8, 8)
    pltpu.sync_copy(a_hbm.at[pl.ds(ast, RNG * 8)], atab.at[pl.ds(0, RNG * 8)])

    def istep(i, _):
        htab[pl.ds(i * 16, 16)] = jnp.full((16,), jnp.inf, jnp.float32)
        return 0

    lax.fori_loop(0, TFL // 16, istep, 0)

    pltpu.sync_copy(cnts_hbm, cn_v)
    ex0, ex1, seg0, seg1, _p0, _p1v = _offsets(cn_v, r)
    r15 = jnp.bitwise_and(r, 15)
    ss = pl.multiple_of(_extract(jnp.where(r < 16, ex0, ex1), r15), 8)
    slen = _extract(jnp.where(r < 16, seg0, seg1), r15)
    nwin = (slen + WE - 1) // WE

    iota = lax.iota(jnp.int32, 16)
    iota8 = iota * 8

    def win(w, _):
        wst = pl.multiple_of(ss + w * WE, 8)
        pltpu.sync_copy(bdst.at[pl.ds(wst, WE)], dwin)
        pltpu.sync_copy(bsrc.at[pl.ds(wst, WE)], swin)
        pltpu.sync_copy(bea.at[pl.ds(wst, WE)], eawin)
        pltpu.async_copy(b_hbm.at[swin], brows, semg).wait()

        def step(i, _):
            sl = pl.ds(i * 16, 16)
            rel = dwin[sl]
            eav = eawin[sl]
            msk = (iota + (w * WE + i * 16)) < slen
            i8 = rel * 8
            erow = iota + i * 16
            colz = jnp.zeros((16,), jnp.int32)
            a = [plsc.load_gather(atab, [i8 + f]) for f in range(8)]
            b = [plsc.load_gather(brows, [erow, colz + f]) for f in range(8)]
            m = []
            h = []
            for f in range(8):
                pre = a[f] + b[f] + eav * wc[f]
                h.append(jnp.maximum(pre, 0.01 * pre))
            for k in range(8):
                acc = jnp.full((16,), 0.0, jnp.float32) + b2[k]
                for f in range(8):
                    acc = acc + h[f] * w2[k * 8 + f]
                m.append(acc)
            rank, _lm = plsc.scan_count(rel)
            mrank = jnp.max(jnp.where(msk, rank, 0))
            mc1 = jnp.logical_and(msk, rank == 1)
            for k in range(8):
                old = plsc.load_gather(htab, [i8 + k], mask=mc1)
                plsc.store_scatter(htab, [i8 + k], jnp.minimum(old, m[k]), mask=mc1)

            @pl.when(mrank >= 2)
            def _():
                def rmw(c, _):
                    mc = jnp.logical_and(msk, rank == (c + 2))
                    for k in range(8):
                        old = plsc.load_gather(htab, [i8 + k], mask=mc)
                        plsc.store_scatter(htab, [i8 + k], jnp.minimum(old, m[k]), mask=mc)
                    return 0

                lax.fori_loop(0, mrank - 1, rmw, 0)

            return 0

        lax.fori_loop(0, WE // 16, step, 0)
        return 0

    lax.fori_loop(0, nwin, win, 0)

    def astep(i, _):
        sl = pl.ds(i * 16, 16)
        v = htab[sl]
        v = jnp.where(v == jnp.inf, 0.0, v)
        htab[sl] = jnp.maximum(v, 0.01 * v)
        return 0

    lax.fori_loop(0, TFL // 16, astep, 0)
    pltpu.sync_copy(htab.at[pl.ds(0, RNG * 8)], act_hbm.at[pl.ds(ast, RNG * 8)])

    _node_tables_2d(htab, atab, brows, b2_hbm, a2_hbm, r, w1a, w1b, b1)


def _node_tables_2d(src_tab, atab, brow2d, b_hbm, a_hbm, r, w1a, w1b, b1):
    iota = lax.iota(jnp.int32, 16)
    half = 1568

    for p in range(2):
        nbase = p * half
        nrows = half if p == 0 else RNG - half
        nsteps = (nrows + 15) // 16

        def pstep(i, _):
            n = iota + nbase + i * 16
            nc = jnp.minimum(n, RNG)
            msk = n < (nbase + nrows)
            n8 = nc * 8
            actf = [plsc.load_gather(src_tab, [n8 + f]) for f in range(8)]
            nrel = nc - nbase
            for g in range(8):
                ag = jnp.full((16,), 0.0, jnp.float32) + b1[g]
                bg = jnp.zeros((16,), jnp.float32)
                for f in range(8):
                    ag = ag + actf[f] * w1a[g * 8 + f]
                    bg = bg + actf[f] * w1b[g * 8 + f]
                plsc.store_scatter(atab, [n8 + g], ag, mask=msk)
                plsc.store_scatter(brow2d, [nrel, jnp.full((16,), g, jnp.int32)], bg, mask=msk)
            return 0

        lax.fori_loop(0, nsteps, pstep, 0)
        dstst = pl.multiple_of(r * RNG + nbase, 1)
        pltpu.sync_copy(brow2d.at[pl.ds(0, nrows)], b_hbm.at[pl.ds(dstst, nrows)])

    ast = pl.multiple_of(r * RNG * 8, 8)
    pltpu.sync_copy(atab.at[pl.ds(0, RNG * 8)], a_hbm.at[pl.ds(ast, RNG * 8)])


def _final_body(x_ref, h_ref, wla_ref, wlb_ref, bl_ref, o_ref):
    x0 = jnp.trunc(x_ref[...])
    z = (jnp.sum(x0 * wla_ref[...], axis=1, keepdims=True)
         + jnp.sum(h_ref[...] * wlb_ref[...], axis=1, keepdims=True) + bl_ref[...])
    o_ref[...] = jnp.maximum(z, 0.0) + jnp.log1p(jnp.exp(-jnp.abs(z)))


def kernel(node_features, edge_features, W11, b11, W12, b12, W21, b21, W22, b22,
           W31, b31, W32, b32, Wl, bl, edge_index):
    B, n, F = node_features.shape
    nf = node_features[0]
    src = edge_index[0, 0]
    dst = edge_index[0, 1]
    eaf = edge_features[0, :, 0]

    def pw(W1, b1):
        return jnp.concatenate([W1[:, :8].reshape(-1), W1[:, 8:16].reshape(-1), b1])

    def ew(W1, W2, b2):
        return jnp.concatenate([W2.reshape(-1), W1[:, 16], b2])

    pw1 = pw(W11, b11)
    pw2 = pw(W21, b21)
    pw3 = pw(W31, b31)
    ew1 = ew(W11, W12, b12)
    ew2 = ew(W21, W22, b22)
    ew3 = ew(W31, W32, b32)

    cnts = _prep_a(dst)
    bdst, bsrc, bea = _prep_b(dst, src, eaf, cnts)
    a1, b1t = _p1(nf.reshape(-1), pw1)
    _act1, a2, b2t = _edge(bdst, bsrc, bea, cnts, a1, b1t, ew1, pw2)
    _act2, a3, b3t = _edge(bdst, bsrc, bea, cnts, a2, b2t, ew2, pw3)
    act3, _a4, _b4 = _edge(bdst, bsrc, bea, cnts, a3, b3t, ew3, pw3)

    BN = 10000
    wla = Wl[:, :F]
    wlb = Wl[:, F:]
    out = pl.pallas_call(
        _final_body,
        grid=(N // BN,),
        in_specs=[
            pl.BlockSpec((BN, F), lambda i: (i, 0)),
            pl.BlockSpec((BN, 8), lambda i: (i, 0)),
            pl.BlockSpec(wla.shape, lambda i: (0, 0)),
            pl.BlockSpec(wlb.shape, lambda i: (0, 0)),
            pl.BlockSpec((1, 1), lambda i: (0, 0)),
        ],
        out_specs=pl.BlockSpec((BN, 1), lambda i: (i, 0)),
        out_shape=jax.ShapeDtypeStruct((N, 1), jnp.float32),
    )(nf, act3.reshape(N, 8), wla, wlb, bl.reshape(1, 1))
    return out.reshape(B, n, 1)
